# baseline (speedup 1.0000x reference)
import functools

import jax
import jax.numpy as jnp
import numpy as np
from jax import lax
from jax.experimental import pallas as pl
from jax.experimental.pallas import tpu as pltpu
from jax.experimental.pallas import tpu_sc as plsc

_NUM_OBJ = 4
_NUM_LEVELS = 16
_FPL = 2
_T = 1 << 19
_BASE_RES = 16
_GROWTH = 1.3819
_N = 524288

_PY = np.int32(np.uint32(2654435761).view(np.int32))
_PZ = np.int32(805459861)


def _levels():
    ress, sizes, offsets = [], [], []
    off = 0
    for l in range(_NUM_LEVELS):
        res = int(np.floor(_BASE_RES * (_GROWTH ** l)))
        nv = (res + 1) ** 3
        size = min(nv, _T)
        ress.append(res)
        sizes.append(size)
        offsets.append(off)
        off += size
    return ress, sizes, offsets, off


_RESS, _SIZES, _OFFSETS, _TOTAL_ROWS = _levels()

_NC, _NS = 2, 16
_NW = _NC * _NS
_C = 128
_PPW = _N // _NW
_CHUNKS = _PPW // _C
_NF = _NUM_LEVELS * _FPL

_CORNERS = [(dx, dy, dz) for dx in (0, 1) for dy in (0, 1) for dz in (0, 1)]


def _sc_body(xs_hbm, ys_hbm, zs_hbm, ob_hbm, tab_hbm, out_hbm,
             px, py, pz, obj_v, idx_a, idx_b, rows_a, rows_b, out_v,
             sem_a, sem_b):
    wid = lax.axis_index("s") * _NC + lax.axis_index("c")
    pltpu.sync_copy(ob_hbm, obj_v)
    obase2 = obj_v[...]
    iota = lax.iota(jnp.int32, 16)

    def compute_idx(l, idx_buf):
        res, size, off = _RESS[l], _SIZES[l], _OFFSETS[l]
        dense = size == (res + 1) ** 3
        base_l2 = obase2 + 2 * off
        fr = jnp.float32(res)

        def g_body(g, c_):
            s = g * 16
            x = px[pl.ds(s, 16)]
            y = py[pl.ds(s, 16)]
            z = pz[pl.ds(s, 16)]
            xi = (x * fr).astype(jnp.int32)
            yi = (y * fr).astype(jnp.int32)
            zi = (z * fr).astype(jnp.int32)
            cx = jnp.minimum(xi, res - 1)
            cy = jnp.minimum(yi, res - 1)
            cz = jnp.minimum(zi, res - 1)
            if dense:
                r1 = res + 1
                a00 = cy + r1 * cz
                a01 = a00 + r1
                a10 = a00 + 1
                a11 = a01 + 1
                ra = {(0, 0): r1 * a00, (0, 1): r1 * a01,
                      (1, 0): r1 * a10, (1, 1): r1 * a11}
                for c, (dx, dy, dz) in enumerate(_CORNERS):
                    core = (cx + dx) + ra[(dy, dz)]
                    w0 = core * 2 + base_l2
                    idx_buf[2 * c, pl.ds(s, 16)] = w0
                    idx_buf[2 * c + 1, pl.ds(s, 16)] = w0 + 1
            else:
                mask = size - 1
                hx0, hx1 = cx, cx + 1
                hy0 = cy * _PY
                hy1 = hy0 + _PY
                hz0 = cz * _PZ
                hz1 = hz0 + _PZ
                e = {(0, 0): hy0 ^ hz0, (0, 1): hy0 ^ hz1,
                     (1, 0): hy1 ^ hz0, (1, 1): hy1 ^ hz1}
                for c, (dx, dy, dz) in enumerate(_CORNERS):
                    h = (hx1 if dx else hx0) ^ e[(dy, dz)]
                    w0 = (h & mask) * 2 + base_l2
                    idx_buf[2 * c, pl.ds(s, 16)] = w0
                    idx_buf[2 * c + 1, pl.ds(s, 16)] = w0 + 1
            return c_

        lax.fori_loop(0, _C // 16, g_body, 0)

    def issue(idx_buf, rows_buf, sem):
        return [
            pltpu.async_copy(tab_hbm.at[idx_buf.at[cf]],
                             rows_buf.at[pl.ds(cf * _C, _C)], sem)
            for cf in range(16)
        ]

    def accumulate(l, rows_buf):
        fr = jnp.float32(_RESS[l])

        def g_body(g, c_):
            s = g * 16
            x = px[pl.ds(s, 16)]
            y = py[pl.ds(s, 16)]
            z = pz[pl.ds(s, 16)]
            xf, yf, zf = x * fr, y * fr, z * fr
            wx = xf - xf.astype(jnp.int32).astype(jnp.float32)
            wy = yf - yf.astype(jnp.int32).astype(jnp.float32)
            wz = zf - zf.astype(jnp.int32).astype(jnp.float32)
            cwx, cwy, cwz = 1.0 - wx, 1.0 - wy, 1.0 - wz
            wyz = {(0, 0): cwy * cwz, (0, 1): cwy * wz,
                   (1, 0): wy * cwz, (1, 1): wy * wz}
            acc0 = jnp.zeros((16,), jnp.float32)
            acc1 = jnp.zeros((16,), jnp.float32)
            for c, (dx, dy, dz) in enumerate(_CORNERS):
                wt = (wx if dx else cwx) * wyz[(dy, dz)]
                v0 = rows_buf[pl.ds(2 * c * _C + s, 16)]
                v1 = rows_buf[pl.ds((2 * c + 1) * _C + s, 16)]
                acc0 = acc0 + wt * v0
                acc1 = acc1 + wt * v1
            out_v[2 * l, pl.ds(s, 16)] = acc0
            out_v[2 * l + 1, pl.ds(s, 16)] = acc1
            return c_

        lax.fori_loop(0, _C // 16, g_body, 0)

    def chunk_body(ck, carry):
        base = wid * _PPW + ck * _C
        ckg = wid * _CHUNKS + ck
        pltpu.sync_copy(xs_hbm.at[pl.ds(base, _C)], px)
        pltpu.sync_copy(ys_hbm.at[pl.ds(base, _C)], py)
        pltpu.sync_copy(zs_hbm.at[pl.ds(base, _C)], pz)

        compute_idx(0, idx_a)
        hs = issue(idx_a, rows_a, sem_a)
        for l in range(1, _NUM_LEVELS):
            ib, rb, sm = (idx_b, rows_b, sem_b) if l % 2 else (idx_a, rows_a, sem_a)
            compute_idx(l, ib)
            hs_new = issue(ib, rb, sm)
            for h in hs:
                h.wait()
            accumulate(l - 1, rows_b if (l - 1) % 2 else rows_a)
            hs = hs_new
        for h in hs:
            h.wait()
        accumulate(_NUM_LEVELS - 1, rows_b if (_NUM_LEVELS - 1) % 2 else rows_a)

        pltpu.sync_copy(out_v, out_hbm.at[ckg])
        return carry

    lax.fori_loop(0, _CHUNKS, chunk_body, 0)


_hashgrid_sc = functools.partial(
    pl.kernel,
    out_type=jax.ShapeDtypeStruct((_NW * _CHUNKS, _NF, _C), jnp.float32),
    mesh=plsc.VectorSubcoreMesh(core_axis_name="c", subcore_axis_name="s",
                                num_cores=_NC, num_subcores=_NS),
    scratch_types=[
        pltpu.VMEM((_C,), jnp.float32),
        pltpu.VMEM((_C,), jnp.float32),
        pltpu.VMEM((_C,), jnp.float32),
        pltpu.VMEM((16,), jnp.int32),
        pltpu.VMEM((16, _C), jnp.int32),
        pltpu.VMEM((16, _C), jnp.int32),
        pltpu.VMEM((16 * _C,), jnp.float32),
        pltpu.VMEM((16 * _C,), jnp.float32),
        pltpu.VMEM((_NF, _C), jnp.float32),
        pltpu.SemaphoreType.DMA,
        pltpu.SemaphoreType.DMA,
    ],
)(_sc_body)


def kernel(positions_flat, obj_id, tables):
    xs = positions_flat[:, 0]
    ys = positions_flat[:, 1]
    zs = positions_flat[:, 2]
    ob = jnp.full((16,), jnp.asarray(obj_id, jnp.int32) * (2 * _TOTAL_ROWS),
                  jnp.int32)
    tab = tables.reshape(_NUM_OBJ * _TOTAL_ROWS * _FPL)
    out = _hashgrid_sc(xs, ys, zs, ob, tab)
    return out.transpose(0, 2, 1).reshape(_N, _NF)
